# Initial kernel scaffold; baseline (speedup 1.0000x reference)
import jax
import jax.numpy as jnp
from jax.experimental import pallas as pl


def kernel(x, edge_index, W1, b1, W2, b2):
    raise NotImplementedError("write your pallas kernel here")


# speedup vs baseline: 8.0128x; 8.0128x over previous
import functools

import jax
import jax.numpy as jnp
from jax import lax
from jax.experimental import pallas as pl
from jax.experimental.pallas import tpu as pltpu
from jax.experimental.pallas import tpu_sc as plsc

NC, NS = 2, 16
NW = NC * NS
LANES = 16
CHUNK = 128
BM = 512


def _sc_mesh():
    return plsc.VectorSubcoreMesh(
        core_axis_name="c", subcore_axis_name="s",
        num_cores=NC, num_subcores=NS)


def _make_deg_kernel(n_pad, k_chunks, d):
    rows_per_sub = n_pad // NS
    assert rows_per_sub % 64 == 0

    @functools.partial(
        pl.kernel,
        out_type=jax.ShapeDtypeStruct((NC, n_pad, d), jnp.float32),
        mesh=_sc_mesh(),
        scratch_types=[
            pltpu.VMEM((k_chunks, CHUNK), jnp.int32),
            pltpu.VMEM((CHUNK, d), jnp.float32),
            pltpu.VMEM((64, d), jnp.float32),
            pltpu.VMEM_SHARED((n_pad, d), jnp.float32),
        ],
    )
    def deg_kernel(dst_hbm, out_hbm, idx_v, ones_v, zb_v, acc_sh):
        cid = lax.axis_index("c")
        sid = lax.axis_index("s")
        w = cid * NS + sid

        @pl.loop(0, CHUNK)
        def _(r):
            @pl.loop(0, d, step=LANES)
            def _(c):
                ones_v[r, pl.ds(c, LANES)] = jnp.full(
                    (LANES,), 1.0, jnp.float32)

        @pl.loop(0, 64)
        def _(r):
            @pl.loop(0, d, step=LANES)
            def _(c):
                zb_v[r, pl.ds(c, LANES)] = jnp.zeros((LANES,), jnp.float32)

        @pl.loop(0, rows_per_sub // 64)
        def _(t):
            pltpu.sync_copy(
                zb_v, acc_sh.at[pl.ds(sid * rows_per_sub + t * 64, 64)])

        plsc.subcore_barrier()
        pltpu.sync_copy(dst_hbm.at[w], idx_v)

        @pl.loop(0, k_chunks)
        def _(j):
            pltpu.sync_copy(ones_v, acc_sh.at[idx_v.at[j]], add=True)

        plsc.subcore_barrier()
        pltpu.sync_copy(
            acc_sh.at[pl.ds(sid * rows_per_sub, rows_per_sub)],
            out_hbm.at[cid, pl.ds(sid * rows_per_sub, rows_per_sub)])

    return deg_kernel


def _make_agg_kernel(n_pad, k_chunks, d):
    rows_per_sub = n_pad // NS
    assert rows_per_sub % 64 == 0

    @functools.partial(
        pl.kernel,
        out_type=jax.ShapeDtypeStruct((NC, n_pad, d), jnp.float32),
        mesh=_sc_mesh(),
        scratch_types=[
            pltpu.VMEM((k_chunks, CHUNK), jnp.int32),
            pltpu.VMEM((k_chunks, CHUNK), jnp.int32),
            pltpu.VMEM((CHUNK, d), jnp.float32),
            pltpu.VMEM((64, d), jnp.float32),
            pltpu.VMEM_SHARED((n_pad, d), jnp.float32),
        ],
    )
    def agg_kernel(table_hbm, src_hbm, dst_hbm, out_hbm,
                   si_v, di_v, rows_v, zb_v, acc_sh):
        cid = lax.axis_index("c")
        sid = lax.axis_index("s")
        w = cid * NS + sid

        @pl.loop(0, 64)
        def _(r):
            @pl.loop(0, d, step=LANES)
            def _(c):
                zb_v[r, pl.ds(c, LANES)] = jnp.zeros((LANES,), jnp.float32)

        @pl.loop(0, rows_per_sub // 64)
        def _(t):
            pltpu.sync_copy(
                zb_v, acc_sh.at[pl.ds(sid * rows_per_sub + t * 64, 64)])

        plsc.subcore_barrier()
        pltpu.sync_copy(src_hbm.at[w], si_v)
        pltpu.sync_copy(dst_hbm.at[w], di_v)

        @pl.loop(0, k_chunks)
        def _(j):
            pltpu.sync_copy(table_hbm.at[si_v.at[j]], rows_v)
            pltpu.sync_copy(rows_v, acc_sh.at[di_v.at[j]], add=True)

        plsc.subcore_barrier()
        pltpu.sync_copy(
            acc_sh.at[pl.ds(sid * rows_per_sub, rows_per_sub)],
            out_hbm.at[cid, pl.ds(sid * rows_per_sub, rows_per_sub)])

    return agg_kernel


def _tc_matmul(x_p, W):
    n_pad, d_in = x_p.shape
    d_o = W.shape[1]

    def body(x_ref, w_ref, o_ref):
        o_ref[...] = jnp.dot(x_ref[...], w_ref[...],
                             preferred_element_type=jnp.float32)

    return pl.pallas_call(
        body,
        grid=(n_pad // BM,),
        in_specs=[pl.BlockSpec((BM, d_in), lambda i: (i, 0)),
                  pl.BlockSpec((d_in, d_o), lambda i: (0, 0))],
        out_specs=pl.BlockSpec((BM, d_o), lambda i: (i, 0)),
        out_shape=jax.ShapeDtypeStruct((n_pad, d_o), jnp.float32),
    )(x_p, W)


def _dinv_block(da_ref, db_ref):
    deg = da_ref[...] + db_ref[...] + 1.0
    return lax.rsqrt(deg)


def _tc_scale(h, dega, degb):
    n_pad, d = h.shape

    def body(h_ref, da_ref, db_ref, o_ref):
        o_ref[...] = _dinv_block(da_ref, db_ref) * h_ref[...]

    return pl.pallas_call(
        body,
        grid=(n_pad // BM,),
        in_specs=[pl.BlockSpec((BM, d), lambda i: (i, 0)),
                  pl.BlockSpec((BM, d), lambda i: (i, 0)),
                  pl.BlockSpec((BM, d), lambda i: (i, 0))],
        out_specs=pl.BlockSpec((BM, d), lambda i: (i, 0)),
        out_shape=jax.ShapeDtypeStruct((n_pad, d), jnp.float32),
    )(h, dega, degb)


def _tc_mid(agg_a, agg_b, ht, dega, degb, b1):
    n_pad, d = ht.shape

    def body(aa_ref, ab_ref, ht_ref, da_ref, db_ref, b_ref, o_ref):
        dinv = _dinv_block(da_ref, db_ref)
        z = dinv * (aa_ref[...] + ab_ref[...] + ht_ref[...]) + b_ref[...]
        z = jnp.maximum(z, 0.0)
        o_ref[...] = dinv * z

    return pl.pallas_call(
        body,
        grid=(n_pad // BM,),
        in_specs=[pl.BlockSpec((BM, d), lambda i: (i, 0)),
                  pl.BlockSpec((BM, d), lambda i: (i, 0)),
                  pl.BlockSpec((BM, d), lambda i: (i, 0)),
                  pl.BlockSpec((BM, d), lambda i: (i, 0)),
                  pl.BlockSpec((BM, d), lambda i: (i, 0)),
                  pl.BlockSpec((1, d), lambda i: (0, 0))],
        out_specs=pl.BlockSpec((BM, d), lambda i: (i, 0)),
        out_shape=jax.ShapeDtypeStruct((n_pad, d), jnp.float32),
    )(agg_a, agg_b, ht, dega, degb, b1)


def _tc_out(agg_a, agg_b, ht, dega, degb, W2, b2):
    n_pad, d = ht.shape
    d_o = W2.shape[1]

    def body(aa_ref, ab_ref, ht_ref, da_ref, db_ref, w_ref, b_ref, o_ref):
        dinv = _dinv_block(da_ref, db_ref)
        pre = dinv * (aa_ref[...] + ab_ref[...] + ht_ref[...])
        acc = jnp.dot(pre, w_ref[...], preferred_element_type=jnp.float32)
        o_ref[...] = jnp.maximum(acc + b_ref[...], 0.0)

    return pl.pallas_call(
        body,
        grid=(n_pad // BM,),
        in_specs=[pl.BlockSpec((BM, d), lambda i: (i, 0)),
                  pl.BlockSpec((BM, d), lambda i: (i, 0)),
                  pl.BlockSpec((BM, d), lambda i: (i, 0)),
                  pl.BlockSpec((BM, d), lambda i: (i, 0)),
                  pl.BlockSpec((BM, d), lambda i: (i, 0)),
                  pl.BlockSpec((d, d_o), lambda i: (0, 0)),
                  pl.BlockSpec((1, d_o), lambda i: (0, 0))],
        out_specs=pl.BlockSpec((BM, d_o), lambda i: (i, 0)),
        out_shape=jax.ShapeDtypeStruct((n_pad, d_o), jnp.float32),
    )(agg_a, agg_b, ht, dega, degb, W2, b2)


def kernel(x, edge_index, W1, b1, W2, b2):
    N, d_in = x.shape
    d_hid = W1.shape[1]
    d_out = W2.shape[1]
    E = edge_index.shape[1]

    k_chunks = -(-E // (NW * CHUNK))
    e_pad = NW * k_chunks * CHUNK
    n_pad = -(-max(N + 1, BM) // (NS * 64)) * (NS * 64)

    src = edge_index[0].astype(jnp.int32)
    dst = edge_index[1].astype(jnp.int32)
    if e_pad > E:
        fill = jnp.full((e_pad - E,), N, jnp.int32)
        src = jnp.concatenate([src, fill])
        dst = jnp.concatenate([dst, fill])
    src_r = src.reshape(NW, k_chunks, CHUNK)
    dst_r = dst.reshape(NW, k_chunks, CHUNK)
    x_p = jnp.pad(x, ((0, n_pad - N), (0, 0)))
    b1r = b1.reshape(1, d_hid)
    b2r = b2.reshape(1, d_out)

    deg_fn = _make_deg_kernel(n_pad, k_chunks, d_hid)
    agg_fn = _make_agg_kernel(n_pad, k_chunks, d_hid)

    degc = deg_fn(dst_r)
    h1 = _tc_matmul(x_p, W1)
    dega, degb = degc[0], degc[1]
    ht1 = _tc_scale(h1, dega, degb)
    agg1 = agg_fn(ht1, src_r, dst_r)
    htz = _tc_mid(agg1[0], agg1[1], ht1, dega, degb, b1r)
    agg2 = agg_fn(htz, src_r, dst_r)
    out = _tc_out(agg2[0], agg2[1], htz, dega, degb, W2, b2r)
    return out[:N]


# speedup vs baseline: 8.5812x; 1.0709x over previous
import functools

import jax
import jax.numpy as jnp
from jax import lax
from jax.experimental import pallas as pl
from jax.experimental.pallas import tpu as pltpu
from jax.experimental.pallas import tpu_sc as plsc

NC, NS = 2, 16
NW = NC * NS
LANES = 16
CHUNK = 128
BM = 512


def _sc_mesh():
    return plsc.VectorSubcoreMesh(
        core_axis_name="c", subcore_axis_name="s",
        num_cores=NC, num_subcores=NS)


def _make_deg_kernel(n_pad, k_chunks, d):
    rows_per_sub = n_pad // NS
    assert rows_per_sub % 64 == 0

    @functools.partial(
        pl.kernel,
        out_type=jax.ShapeDtypeStruct((NC, n_pad, d), jnp.float32),
        mesh=_sc_mesh(),
        scratch_types=[
            pltpu.VMEM((k_chunks, CHUNK), jnp.int32),
            pltpu.VMEM((CHUNK, d), jnp.float32),
            pltpu.VMEM((64, d), jnp.float32),
            pltpu.VMEM_SHARED((n_pad, d), jnp.float32),
        ],
    )
    def deg_kernel(dst_hbm, out_hbm, idx_v, ones_v, zb_v, acc_sh):
        cid = lax.axis_index("c")
        sid = lax.axis_index("s")
        w = cid * NS + sid

        @pl.loop(0, CHUNK)
        def _(r):
            @pl.loop(0, d, step=LANES)
            def _(c):
                ones_v[r, pl.ds(c, LANES)] = jnp.full(
                    (LANES,), 1.0, jnp.float32)

        @pl.loop(0, 64)
        def _(r):
            @pl.loop(0, d, step=LANES)
            def _(c):
                zb_v[r, pl.ds(c, LANES)] = jnp.zeros((LANES,), jnp.float32)

        @pl.loop(0, rows_per_sub // 64)
        def _(t):
            pltpu.sync_copy(
                zb_v, acc_sh.at[pl.ds(sid * rows_per_sub + t * 64, 64)])

        plsc.subcore_barrier()
        pltpu.sync_copy(dst_hbm.at[w], idx_v)

        @pl.loop(0, k_chunks)
        def _(j):
            pltpu.sync_copy(ones_v, acc_sh.at[idx_v.at[j]], add=True)

        plsc.subcore_barrier()
        pltpu.sync_copy(
            acc_sh.at[pl.ds(sid * rows_per_sub, rows_per_sub)],
            out_hbm.at[cid, pl.ds(sid * rows_per_sub, rows_per_sub)])

    return deg_kernel


def _make_agg_kernel(n_pad, k_chunks, d):
    rows_per_sub = n_pad // NS
    assert rows_per_sub % CHUNK == 0
    assert k_chunks % 2 == 0

    @functools.partial(
        pl.kernel,
        out_type=jax.ShapeDtypeStruct((NC, n_pad, d), jnp.float32),
        mesh=_sc_mesh(),
        scratch_types=[
            pltpu.VMEM((k_chunks, CHUNK), jnp.int32),
            pltpu.VMEM((k_chunks, CHUNK), jnp.int32),
            pltpu.VMEM((CHUNK, d), jnp.float32),
            pltpu.VMEM((CHUNK, d), jnp.float32),
            pltpu.VMEM_SHARED((n_pad, d), jnp.float32),
            pltpu.SemaphoreType.DMA,
            pltpu.SemaphoreType.DMA,
        ],
    )
    def agg_kernel(table_hbm, src_hbm, dst_hbm, out_hbm,
                   si_v, di_v, rows0_v, rows1_v, acc_sh, sem0, sem1):
        cid = lax.axis_index("c")
        sid = lax.axis_index("s")
        w = cid * NS + sid

        @pl.loop(0, CHUNK)
        def _(r):
            @pl.loop(0, d, step=LANES)
            def _(c):
                rows0_v[r, pl.ds(c, LANES)] = jnp.zeros((LANES,), jnp.float32)

        @pl.loop(0, rows_per_sub // CHUNK)
        def _(t):
            pltpu.sync_copy(
                rows0_v,
                acc_sh.at[pl.ds(sid * rows_per_sub + t * CHUNK, CHUNK)])

        plsc.subcore_barrier()
        pltpu.sync_copy(src_hbm.at[w], si_v)
        pltpu.sync_copy(dst_hbm.at[w], di_v)

        pltpu.async_copy(table_hbm.at[si_v.at[0]], rows0_v, sem0)

        @pl.loop(0, k_chunks, step=2)
        def _(j):
            pltpu.make_async_copy(
                table_hbm.at[si_v.at[j]], rows0_v, sem0).wait()
            pltpu.async_copy(table_hbm.at[si_v.at[j + 1]], rows1_v, sem1)
            pltpu.sync_copy(rows0_v, acc_sh.at[di_v.at[j]], add=True)
            pltpu.make_async_copy(
                table_hbm.at[si_v.at[j + 1]], rows1_v, sem1).wait()

            @pl.when(j + 2 < k_chunks)
            def _():
                pltpu.async_copy(
                    table_hbm.at[si_v.at[j + 2]], rows0_v, sem0)

            pltpu.sync_copy(rows1_v, acc_sh.at[di_v.at[j + 1]], add=True)

        plsc.subcore_barrier()
        pltpu.sync_copy(
            acc_sh.at[pl.ds(sid * rows_per_sub, rows_per_sub)],
            out_hbm.at[cid, pl.ds(sid * rows_per_sub, rows_per_sub)])

    return agg_kernel


def _tc_matmul(x_p, W):
    n_pad, d_in = x_p.shape
    d_o = W.shape[1]

    def body(x_ref, w_ref, o_ref):
        o_ref[...] = jnp.dot(x_ref[...], w_ref[...],
                             preferred_element_type=jnp.float32)

    return pl.pallas_call(
        body,
        grid=(n_pad // BM,),
        in_specs=[pl.BlockSpec((BM, d_in), lambda i: (i, 0)),
                  pl.BlockSpec((d_in, d_o), lambda i: (0, 0))],
        out_specs=pl.BlockSpec((BM, d_o), lambda i: (i, 0)),
        out_shape=jax.ShapeDtypeStruct((n_pad, d_o), jnp.float32),
    )(x_p, W)


def _dinv_block(da_ref, db_ref):
    deg = da_ref[...] + db_ref[...] + 1.0
    return lax.rsqrt(deg)


def _tc_scale(h, dega, degb):
    n_pad, d = h.shape

    def body(h_ref, da_ref, db_ref, o_ref):
        o_ref[...] = _dinv_block(da_ref, db_ref) * h_ref[...]

    return pl.pallas_call(
        body,
        grid=(n_pad // BM,),
        in_specs=[pl.BlockSpec((BM, d), lambda i: (i, 0)),
                  pl.BlockSpec((BM, d), lambda i: (i, 0)),
                  pl.BlockSpec((BM, d), lambda i: (i, 0))],
        out_specs=pl.BlockSpec((BM, d), lambda i: (i, 0)),
        out_shape=jax.ShapeDtypeStruct((n_pad, d), jnp.float32),
    )(h, dega, degb)


def _tc_mid(agg_a, agg_b, ht, dega, degb, b1):
    n_pad, d = ht.shape

    def body(aa_ref, ab_ref, ht_ref, da_ref, db_ref, b_ref, o_ref):
        dinv = _dinv_block(da_ref, db_ref)
        z = dinv * (aa_ref[...] + ab_ref[...] + ht_ref[...]) + b_ref[...]
        z = jnp.maximum(z, 0.0)
        o_ref[...] = dinv * z

    return pl.pallas_call(
        body,
        grid=(n_pad // BM,),
        in_specs=[pl.BlockSpec((BM, d), lambda i: (i, 0)),
                  pl.BlockSpec((BM, d), lambda i: (i, 0)),
                  pl.BlockSpec((BM, d), lambda i: (i, 0)),
                  pl.BlockSpec((BM, d), lambda i: (i, 0)),
                  pl.BlockSpec((BM, d), lambda i: (i, 0)),
                  pl.BlockSpec((1, d), lambda i: (0, 0))],
        out_specs=pl.BlockSpec((BM, d), lambda i: (i, 0)),
        out_shape=jax.ShapeDtypeStruct((n_pad, d), jnp.float32),
    )(agg_a, agg_b, ht, dega, degb, b1)


def _tc_out(agg_a, agg_b, ht, dega, degb, W2, b2):
    n_pad, d = ht.shape
    d_o = W2.shape[1]

    def body(aa_ref, ab_ref, ht_ref, da_ref, db_ref, w_ref, b_ref, o_ref):
        dinv = _dinv_block(da_ref, db_ref)
        pre = dinv * (aa_ref[...] + ab_ref[...] + ht_ref[...])
        acc = jnp.dot(pre, w_ref[...], preferred_element_type=jnp.float32)
        o_ref[...] = jnp.maximum(acc + b_ref[...], 0.0)

    return pl.pallas_call(
        body,
        grid=(n_pad // BM,),
        in_specs=[pl.BlockSpec((BM, d), lambda i: (i, 0)),
                  pl.BlockSpec((BM, d), lambda i: (i, 0)),
                  pl.BlockSpec((BM, d), lambda i: (i, 0)),
                  pl.BlockSpec((BM, d), lambda i: (i, 0)),
                  pl.BlockSpec((BM, d), lambda i: (i, 0)),
                  pl.BlockSpec((d, d_o), lambda i: (0, 0)),
                  pl.BlockSpec((1, d_o), lambda i: (0, 0))],
        out_specs=pl.BlockSpec((BM, d_o), lambda i: (i, 0)),
        out_shape=jax.ShapeDtypeStruct((n_pad, d_o), jnp.float32),
    )(agg_a, agg_b, ht, dega, degb, W2, b2)


def kernel(x, edge_index, W1, b1, W2, b2):
    N, d_in = x.shape
    d_hid = W1.shape[1]
    d_out = W2.shape[1]
    E = edge_index.shape[1]

    k_chunks = -(-E // (NW * CHUNK))
    k_chunks += k_chunks % 2
    e_pad = NW * k_chunks * CHUNK
    n_pad = -(-max(N + 1, BM) // (NS * 64)) * (NS * 64)

    src = edge_index[0].astype(jnp.int32)
    dst = edge_index[1].astype(jnp.int32)
    if e_pad > E:
        fill = jnp.full((e_pad - E,), N, jnp.int32)
        src = jnp.concatenate([src, fill])
        dst = jnp.concatenate([dst, fill])
    src_r = src.reshape(NW, k_chunks, CHUNK)
    dst_r = dst.reshape(NW, k_chunks, CHUNK)
    x_p = jnp.pad(x, ((0, n_pad - N), (0, 0)))
    b1r = b1.reshape(1, d_hid)
    b2r = b2.reshape(1, d_out)

    deg_fn = _make_deg_kernel(n_pad, k_chunks, d_hid)
    agg_fn = _make_agg_kernel(n_pad, k_chunks, d_hid)

    degc = deg_fn(dst_r)
    h1 = _tc_matmul(x_p, W1)
    dega, degb = degc[0], degc[1]
    ht1 = _tc_scale(h1, dega, degb)
    agg1 = agg_fn(ht1, src_r, dst_r)
    htz = _tc_mid(agg1[0], agg1[1], ht1, dega, degb, b1r)
    agg2 = agg_fn(htz, src_r, dst_r)
    out = _tc_out(agg2[0], agg2[1], htz, dega, degb, W2, b2r)
    return out[:N]
